# Initial kernel scaffold; baseline (speedup 1.0000x reference)
import jax
import jax.numpy as jnp
from jax.experimental import pallas as pl


def kernel(x, edge_index, edge_attr, W1, b1, W2, b2, Wg, bg, Wfc, bfc):
    raise NotImplementedError("write your pallas kernel here")


# speedup vs baseline: 1.6653x; 1.6653x over previous
import functools
import jax
import jax.numpy as jnp
from jax.experimental import pallas as pl


def _mm_kernel(a_ref, b_ref, o_ref):
    o_ref[...] = jnp.dot(a_ref[...], b_ref[...],
                         preferred_element_type=jnp.float32)


def _mm(a, b, bm):
    m, k = a.shape
    n = b.shape[1]
    return pl.pallas_call(
        _mm_kernel,
        grid=(m // bm,),
        in_specs=[
            pl.BlockSpec((bm, k), lambda i: (i, 0)),
            pl.BlockSpec((k, n), lambda i: (0, 0)),
        ],
        out_specs=pl.BlockSpec((bm, n), lambda i: (i, 0)),
        out_shape=jax.ShapeDtypeStruct((m, n), jnp.float32),
    )(a, b)


def kernel(x, edge_index, edge_attr, W1, b1, W2, b2, Wg, bg, Wfc, bfc):
    n, d = x.shape
    e = edge_attr.shape[0]
    src = edge_index[0]
    dst = edge_index[1]

    def conv(h, W, b):
        wa, wb, wc = W[:d], W[d:2 * d], W[2 * d:]
        A = _mm(h, wa - wb, 1000)
        B = _mm(h, wb, 1000)
        C = _mm(edge_attr, wc, 2000) + b
        m = jax.nn.relu(A[dst] + B[src] + C)
        return jnp.zeros((n, m.shape[1]), jnp.float32).at[dst].max(m)

    h1 = conv(x, W1, b1)
    h2 = conv(h1, W2, b2)

    deg = jnp.ones((n,), jnp.float32).at[dst].add(1.0)
    dinv = jax.lax.rsqrt(deg)
    s = jnp.zeros((n,), jnp.float32).at[src].add(dinv[dst])
    w = dinv * (s + dinv)

    pooled = (w[None, :] @ h2) / n
    out = (pooled @ Wg + bg) @ Wfc + bfc
    return out
